# Initial kernel scaffold; baseline (speedup 1.0000x reference)
import jax
import jax.numpy as jnp
from jax.experimental import pallas as pl


def kernel(x, cond, w_gate, fc1_w, fc1_b, fc2_w, fc2_b):
    raise NotImplementedError("write your pallas kernel here")


# speedup vs baseline: 3.0629x; 3.0629x over previous
import functools

import jax
import jax.numpy as jnp
import numpy as np
from jax.experimental import pallas as pl
from jax.experimental.pallas import tpu as pltpu

D = 768
H = 1536
E = 8
T = 2048
K = 2
BT = 512
NT = T // BT
EPS = float(np.finfo(np.float64).eps)


def _moe_body(x_ref, c_ref, wg_ref, w1_ref, b1_ref, w2_ref, b2_ref,
              o_ref, gates_sc):
    e = pl.program_id(1)

    @pl.when(e == 0)
    def _():
        xb = x_ref[...]
        cb = c_ref[...]
        logits = (jnp.dot(xb, wg_ref[:D, :], preferred_element_type=jnp.float32)
                  + jnp.dot(cb, wg_ref[D:, :], preferred_element_type=jnp.float32))
        iota = jax.lax.broadcasted_iota(jnp.int32, (BT, E), 1)
        m1 = jnp.max(logits, axis=1, keepdims=True)
        e1 = jnp.min(jnp.where(logits >= m1, iota, E), axis=1, keepdims=True)
        oh1 = iota == e1
        neg = jnp.where(oh1, -jnp.inf, logits)
        m2 = jnp.max(neg, axis=1, keepdims=True)
        e2 = jnp.min(jnp.where(neg >= m2, iota, E), axis=1, keepdims=True)
        oh2 = iota == e2
        g1 = jax.nn.sigmoid(m1 - m2)
        g2 = 1.0 - g1
        gates_sc[...] = jnp.where(oh1, g1, 0.0) + jnp.where(oh2, g2, 0.0)

    xb = x_ref[...]
    h = jnp.dot(xb, w1_ref[0], preferred_element_type=jnp.float32) + b1_ref[0]
    h = 0.5 * h * (1.0 + jax.lax.erf(h * (1.0 / np.sqrt(2.0))))
    o = jnp.dot(h, w2_ref[0], preferred_element_type=jnp.float32) + b2_ref[0]
    iota = jax.lax.broadcasted_iota(jnp.int32, (BT, E), 1)
    g = jnp.sum(jnp.where(iota == e, gates_sc[...], 0.0), axis=1, keepdims=True)
    contrib = g * jnp.exp(o)

    @pl.when(e == 0)
    def _():
        o_ref[...] = contrib

    @pl.when(e > 0)
    def _():
        o_ref[...] += contrib

    @pl.when(e == E - 1)
    def _():
        acc = o_ref[...]
        o_ref[...] = jnp.log(jnp.where(acc == 0.0, EPS, acc))


@functools.partial(jax.jit)
def kernel(x, cond, w_gate, fc1_w, fc1_b, fc2_w, fc2_b):
    fc1_b = fc1_b.reshape(E, 1, H)
    fc2_b = fc2_b.reshape(E, 1, D)
    return pl.pallas_call(
        _moe_body,
        grid=(NT, E),
        in_specs=[
            pl.BlockSpec((BT, D), lambda i, e: (i, 0)),
            pl.BlockSpec((BT, D), lambda i, e: (i, 0)),
            pl.BlockSpec((2 * D, E), lambda i, e: (0, 0)),
            pl.BlockSpec((1, D, H), lambda i, e: (e, 0, 0)),
            pl.BlockSpec((1, 1, H), lambda i, e: (e, 0, 0)),
            pl.BlockSpec((1, H, D), lambda i, e: (e, 0, 0)),
            pl.BlockSpec((1, 1, D), lambda i, e: (e, 0, 0)),
        ],
        out_specs=pl.BlockSpec((BT, D), lambda i, e: (i, 0)),
        out_shape=jax.ShapeDtypeStruct((T, D), jnp.float32),
        scratch_shapes=[pltpu.VMEM((BT, E), jnp.float32)],
        compiler_params=pltpu.CompilerParams(
            dimension_semantics=("parallel", "arbitrary"),
        ),
    )(x, cond, w_gate, fc1_w, fc1_b, fc2_w, fc2_b)
